# baseline (speedup 1.0000x reference)
import jax
import jax.numpy as jnp
from jax.experimental import pallas as pl
from jax.experimental.pallas import tpu as pltpu

_N = 8388608
_ROWS = 65536
_GRID = 3
_BLOCK_ROWS = 29952


def _body(t_ref, w_ref, o_ref):
    o_ref[...] = w_ref[...]

    @pl.when(pl.program_id(0) == _GRID - 1)
    def _():
        t = t_ref[0]
        tv = jnp.full((8, 128), t, dtype=jnp.float32)
        cond = (t > 500.0) & (t < 2502.54614894971)
        valv = 14.625 * jnp.where(cond, 0.01 * jnp.sin(0.001571 * (-500.0 + tv)), 0.0)
        ridx = jax.lax.broadcasted_iota(jnp.int32, (8, 128), 0)
        cidx = jax.lax.broadcasted_iota(jnp.int32, (8, 128), 1)
        first = (ridx == 0) & (cidx == 0)
        o_ref[0:8, :] = jnp.where(first, valv, w_ref[0:8, :])


def kernel(y, w, c, t):
    w2 = w.reshape(_ROWS, 128)
    t1 = t.reshape(1)
    out = pl.pallas_call(
        _body,
        grid=(_GRID,),
        in_specs=[
            pl.BlockSpec(memory_space=pltpu.SMEM),
            pl.BlockSpec((_BLOCK_ROWS, 128), lambda i: (_GRID - 1 - i, 0)),
        ],
        out_specs=pl.BlockSpec((_BLOCK_ROWS, 128), lambda i: (_GRID - 1 - i, 0)),
        out_shape=jax.ShapeDtypeStruct((_ROWS, 128), jnp.float32),
    )(t1, w2)
    return out.reshape(_N)


# speedup vs baseline: 1.0452x; 1.0452x over previous
import jax
import jax.numpy as jnp
from jax.experimental import pallas as pl
from jax.experimental.pallas import tpu as pltpu

_N = 8388608
_ROWS = 65536
_GRID = 3
_BLOCK_ROWS = 29952


def _body(t_ref, w_ref, o_ref):
    o_ref[...] = w_ref[...]

    @pl.when(pl.program_id(0) == 0)
    def _():
        t = t_ref[0]
        tv = jnp.full((8, 128), t, dtype=jnp.float32)
        cond = (t > 500.0) & (t < 2502.54614894971)
        valv = 14.625 * jnp.where(cond, 0.01 * jnp.sin(0.001571 * (-500.0 + tv)), 0.0)
        ridx = jax.lax.broadcasted_iota(jnp.int32, (8, 128), 0)
        cidx = jax.lax.broadcasted_iota(jnp.int32, (8, 128), 1)
        first = (ridx == 0) & (cidx == 0)
        o_ref[0:8, :] = jnp.where(first, valv, w_ref[0:8, :])


def kernel(y, w, c, t):
    w2 = w.reshape(_ROWS, 128)
    t1 = t.reshape(1)
    out = pl.pallas_call(
        _body,
        grid=(_GRID,),
        in_specs=[
            pl.BlockSpec(memory_space=pltpu.SMEM),
            pl.BlockSpec((_BLOCK_ROWS, 128), lambda i: (i, 0)),
        ],
        out_specs=pl.BlockSpec((_BLOCK_ROWS, 128), lambda i: (i, 0)),
        out_shape=jax.ShapeDtypeStruct((_ROWS, 128), jnp.float32),
    )(t1, w2)
    return out.reshape(_N)
